# baseline (speedup 1.0000x reference)
import jax
import jax.numpy as jnp
from jax import lax
from jax.experimental import pallas as pl
from jax.experimental.pallas import tpu as pltpu

B = 1024
V = 100000
K = 64
TEMP = 10.0

RB = 8
VPAD = 100096
NSCR = 4


def _fused_body(c0_s, lam_ref, dist_ref, tgt_ref, nmt_ref, out_ref, *d_refs):
    zero = jnp.zeros((RB, VPAD + 128), jnp.float32)
    for dr in d_refs:
        dr[...] = zero
    lam = lam_ref[...]

    d = dist_ref[...] * (-1.0 / TEMP)
    m = jnp.max(d, axis=-1, keepdims=True)
    e = jnp.exp(d - m)
    p = (e / jnp.sum(e, axis=-1, keepdims=True)) * lam

    lanes = lax.broadcasted_iota(jnp.int32, (1, 128), 1)

    upds = []
    for r in range(RB):
        trow = tgt_ref[r : r + 1, :]
        tcol = jnp.transpose(trow)
        oh = (lax.rem(tcol, 128) == lanes).astype(jnp.float32)
        mask = (tcol // 128 == trow // 128).astype(jnp.float32)
        mw = mask * p[r : r + 1, :]
        upds.append(jax.lax.dot_general(
            mw, oh, (((1,), (0,)), ((), ())),
            preferred_element_type=jnp.float32,
        ))

    for k in range(K):
        for r in range(RB):
            c0 = pl.multiple_of(c0_s[r, k], 128)
            d_refs[(r + k) % NSCR][r : r + 1, pl.ds(c0, 128)] = (
                upds[r][k : k + 1, :]
            )

    acc = (1.0 - lam) * nmt_ref[...]
    for dr in d_refs:
        acc = acc + dr[:, :V]
    out_ref[...] = acc


_fused = pl.pallas_call(
    _fused_body,
    grid=(B // RB,),
    in_specs=[
        pl.BlockSpec((RB, K), lambda i: (i, 0), memory_space=pltpu.SMEM),
        pl.BlockSpec((RB, 1), lambda i: (i, 0)),
        pl.BlockSpec((RB, K), lambda i: (i, 0)),
        pl.BlockSpec((RB, K), lambda i: (i, 0)),
        pl.BlockSpec((RB, V), lambda i: (i, 0)),
    ],
    out_specs=pl.BlockSpec((RB, V), lambda i: (i, 0)),
    out_shape=jax.ShapeDtypeStruct((B, V), jnp.float32),
    scratch_shapes=[
        pltpu.VMEM((RB, VPAD + 128), jnp.float32) for _ in range(NSCR)
    ],
    compiler_params=pltpu.CompilerParams(
        dimension_semantics=("arbitrary",),
    ),
)


def kernel(nmt_prob, knn_tgt, knn_dist, part_knn_lambda):
    lam2 = part_knn_lambda.reshape(B, 1)
    tgt = knn_tgt.astype(jnp.int32)
    c0 = (tgt // 128) * 128
    eq = c0[:, :, None] == c0[:, None, :]
    lower = jnp.tril(jnp.ones((K, K), jnp.bool_), k=-1)
    dup = jnp.any(eq & lower[None], axis=2)
    c0 = jnp.where(dup, VPAD, c0)
    return _fused(c0, lam2, knn_dist, tgt, nmt_prob)


# speedup vs baseline: 1.1772x; 1.1772x over previous
import jax
import jax.numpy as jnp
from jax import lax
from jax.experimental import pallas as pl
from jax.experimental.pallas import tpu as pltpu

B = 1024
V = 100000
K = 64
TEMP = 10.0

RB = 16
VPAD = 100096


def _fused_body(c0_s, lam_ref, dist_ref, tgt_ref, nmt_ref, out_ref):
    lam = lam_ref[...]
    out_ref[...] = (1.0 - lam) * nmt_ref[...]

    d = dist_ref[...] * (-1.0 / TEMP)
    m = jnp.max(d, axis=-1, keepdims=True)
    e = jnp.exp(d - m)
    p = (e / jnp.sum(e, axis=-1, keepdims=True)) * lam

    lanes = lax.broadcasted_iota(jnp.int32, (1, 128), 1)

    upds = []
    for r in range(RB):
        trow = tgt_ref[r : r + 1, :]
        tcol = jnp.transpose(trow)
        oh = (lax.rem(tcol, 128) == lanes).astype(jnp.float32)
        mask = (tcol // 128 == trow // 128).astype(jnp.float32)
        mw = mask * p[r : r + 1, :]
        upds.append(jax.lax.dot_general(
            mw, oh, (((1,), (0,)), ((), ())),
            preferred_element_type=jnp.float32,
        ))

    oml = 1.0 - lam
    for k in range(K):
        for r in range(RB):
            c0 = pl.multiple_of(c0_s[r, k], 128)
            out_ref[r : r + 1, pl.ds(c0, 128)] = (
                oml[r : r + 1, :] * nmt_ref[r : r + 1, pl.ds(c0, 128)]
                + upds[r][k : k + 1, :]
            )


_fused = pl.pallas_call(
    _fused_body,
    grid=(B // RB,),
    in_specs=[
        pl.BlockSpec((RB, K), lambda i: (i, 0), memory_space=pltpu.SMEM),
        pl.BlockSpec((RB, 1), lambda i: (i, 0)),
        pl.BlockSpec((RB, K), lambda i: (i, 0)),
        pl.BlockSpec((RB, K), lambda i: (i, 0)),
        pl.BlockSpec((RB, VPAD), lambda i: (i, 0)),
    ],
    out_specs=pl.BlockSpec((RB, VPAD), lambda i: (i, 0)),
    out_shape=jax.ShapeDtypeStruct((B, V), jnp.float32),
    compiler_params=pltpu.CompilerParams(
        dimension_semantics=("arbitrary",),
    ),
)


def kernel(nmt_prob, knn_tgt, knn_dist, part_knn_lambda):
    lam2 = part_knn_lambda.reshape(B, 1)
    tgt = knn_tgt.astype(jnp.int32)
    c0 = (tgt // 128) * 128
    return _fused(c0, lam2, knn_dist, tgt, nmt_prob)
